# baseline (speedup 1.0000x reference)
import functools

import jax
import jax.numpy as jnp
from jax import lax
from jax.experimental import pallas as pl
from jax.experimental.pallas import tpu as pltpu, tpu_sc as plsc

B = 16384
F = 26
D = 32
NC = 2
NS = 16
NW = NC * NS
L = 16

ROWS = B * F
RPW = ROWS // NW
PAT = 208
GROUPS = RPW // PAT
QB = 16
QBATCH = 32
QROWS = QBATCH * F
PITCH = D + 1
OROWS = ROWS * D // 128


def _body(idx_hbm, pat_hbm, table_hbm, out_hbm,
          idx_v, pat_v, buf0, buf1, bufp, pbuf, gsem0, gsem1):
    wid = lax.axis_index("s") * NC + lax.axis_index("c")
    base = wid * RPW

    pltpu.sync_copy(idx_hbm.at[pl.ds(base, RPW)], idx_v)
    pltpu.sync_copy(pat_hbm, pat_v)

    pat_regs = [pat_v[pl.ds(j * L, L)] for j in range(PAT // L)]

    def add_group(g, carry):
        s0 = g * PAT
        for j in range(PAT // L):
            sl = pl.ds(s0 + j * L, L)
            idx_v[sl] = idx_v[sl] + pat_regs[j]
        return carry

    lax.fori_loop(0, GROUPS, add_group, 0)

    lanes26 = lax.iota(jnp.int32, L) * F

    def start(q, buf, sem):
        pltpu.async_copy(
            table_hbm.at[idx_v.at[pl.ds(q * QROWS, QROWS)]], buf, sem)

    def wait(buf, sem):
        pltpu.make_async_copy(
            table_hbm.at[idx_v.at[pl.ds(0, QROWS)]], buf, sem).wait()

    def process(q, buf):
        bblk = wid * 4 + lax.div(q, 4)
        bl0 = lax.rem(q, 4) * QBATCH

        def rcopy(r, carry):
            bufp[r, pl.ds(0, L)] = buf[r, pl.ds(0, L)]
            bufp[r, pl.ds(L, L)] = buf[r, pl.ds(L, L)]
            return carry

        lax.fori_loop(0, QROWS, rcopy, 0)

        def fbody(f, carry):
            for c in range(D):
                col = jnp.full((L,), c, jnp.int32)
                for jb in range(QBATCH // L):
                    rows = lanes26 + (jb * L * F + f)
                    val = plsc.load_gather(bufp, [rows, col])
                    pbuf[f * D + c, pl.ds(jb * L, L)] = val
            for cg in range(D // 8):
                orow = f * 4096 + cg * 1024 + bblk * 8
                pltpu.sync_copy(
                    pbuf.at[pl.ds(f * D + cg * 8, 8), :],
                    out_hbm.at[pl.ds(orow, 8), pl.ds(bl0, QBATCH)])
            return carry

        lax.fori_loop(0, F, fbody, 0)

    start(0, buf0, gsem0)

    def loop2(qq, carry):
        q0 = 2 * qq
        q1 = q0 + 1
        start(q1, buf1, gsem1)
        wait(buf0, gsem0)
        process(q0, buf0)

        @pl.when(q0 + 2 < QB)
        def _():
            start(q0 + 2, buf0, gsem0)

        wait(buf1, gsem1)
        process(q1, buf1)
        return carry

    lax.fori_loop(0, QB // 2, loop2, 0)


@jax.jit
def _run(idx_flat, pat, table):
    mesh = plsc.VectorSubcoreMesh(core_axis_name="c", subcore_axis_name="s")
    return pl.kernel(
        _body,
        out_type=jax.ShapeDtypeStruct((OROWS, 128), jnp.float32),
        mesh=mesh,
        scratch_types=[
            pltpu.VMEM((RPW,), jnp.int32),
            pltpu.VMEM((PAT,), jnp.int32),
            pltpu.VMEM((QROWS, D), jnp.float32),
            pltpu.VMEM((QROWS, D), jnp.float32),
            pltpu.VMEM((QROWS, PITCH), jnp.float32),
            pltpu.VMEM((QROWS, QBATCH), jnp.float32),
            pltpu.SemaphoreType.DMA,
            pltpu.SemaphoreType.DMA,
        ],
        compiler_params=pltpu.CompilerParams(
            use_tc_tiling_on_sc=False, needs_layout_passes=False),
    )(idx_flat, pat, table)


def kernel(sparse_indices, offsets, embed_table):
    idx_flat = sparse_indices.reshape(ROWS)
    pat = jnp.tile(offsets.reshape(F), L // 2)
    out = _run(idx_flat, pat, embed_table)
    return (out.reshape(F, 4, 128, 8, 128)
            .transpose(2, 4, 0, 1, 3)
            .reshape(B, F, D))


# speedup vs baseline: 1.0821x; 1.0821x over previous
import functools

import jax
import jax.numpy as jnp
from jax import lax
from jax.experimental import pallas as pl
from jax.experimental.pallas import tpu as pltpu, tpu_sc as plsc

B = 16384
F = 26
D = 32
NC = 2
NS = 16
NW = NC * NS
L = 16

ROWS = B * F
RPW = ROWS // NW
PAT = 208
GROUPS = RPW // PAT
C = 832
NCH = RPW // C


def _body(idx_hbm, pat_hbm, table_hbm, out_hbm,
          idx_v, pat_v, buf0, buf1, gsem, ssem):
    wid = lax.axis_index("s") * NC + lax.axis_index("c")
    base = wid * RPW

    pltpu.sync_copy(idx_hbm.at[pl.ds(base, RPW)], idx_v)
    pltpu.sync_copy(pat_hbm, pat_v)

    pat_regs = [pat_v[pl.ds(j * L, L)] for j in range(PAT // L)]

    def add_group(g, carry):
        s0 = g * PAT
        for j in range(PAT // L):
            sl = pl.ds(s0 + j * L, L)
            idx_v[sl] = idx_v[sl] + pat_regs[j]
        return carry

    lax.fori_loop(0, GROUPS, add_group, 0)

    bufs = [buf0, buf1]
    for k in range(NCH):
        buf = bufs[k % 2]
        pltpu.async_copy(
            table_hbm.at[idx_v.at[pl.ds(k * C, C)]], buf, gsem
        ).wait()
        pltpu.sync_copy(buf, out_hbm.at[pl.ds(base + k * C, C)])


@jax.jit
def _run(idx_flat, pat, table):
    mesh = plsc.VectorSubcoreMesh(core_axis_name="c", subcore_axis_name="s")
    return pl.kernel(
        _body,
        out_type=jax.ShapeDtypeStruct((ROWS, D), jnp.float32),
        mesh=mesh,
        scratch_types=[
            pltpu.VMEM((RPW,), jnp.int32),
            pltpu.VMEM((PAT,), jnp.int32),
            pltpu.VMEM((C, D), jnp.float32),
            pltpu.VMEM((C, D), jnp.float32),
            pltpu.SemaphoreType.DMA,
            pltpu.SemaphoreType.DMA,
        ],
        compiler_params=pltpu.CompilerParams(use_tc_tiling_on_sc=False),
    )(idx_flat, pat, table)


def kernel(sparse_indices, offsets, embed_table):
    idx_flat = sparse_indices.reshape(ROWS)
    pat = jnp.tile(offsets.reshape(F), L // 2)
    out = _run(idx_flat, pat, embed_table)
    return out.reshape(B, F, D)
